# baseline (speedup 1.0000x reference)
import jax
import jax.numpy as jnp
from jax.experimental import pallas as pl

_NUM_CCSK = 8
_SEQ = 128


def _transpose_body(x_ref, o_ref):
    blk = x_ref.shape[0]
    y = jnp.transpose(x_ref[...], (0, 2, 1))
    o_ref[...] = y.reshape(blk, _NUM_CCSK * _SEQ)


def kernel(inputs):
    b, t, f = inputs.shape
    rows = b * t
    x = inputs.reshape(rows, _NUM_CCSK, _SEQ)
    blk = 512
    out = pl.pallas_call(
        _transpose_body,
        grid=(rows // blk,),
        in_specs=[pl.BlockSpec((blk, _NUM_CCSK, _SEQ), lambda i: (i, 0, 0))],
        out_specs=pl.BlockSpec((blk, f), lambda i: (i, 0)),
        out_shape=jax.ShapeDtypeStruct((rows, f), jnp.float32),
    )(x)
    return out.reshape(b, t, f)


# speedup vs baseline: 1.5116x; 1.5116x over previous
import functools

import jax
import jax.numpy as jnp
from jax import lax
from jax.experimental import pallas as pl
from jax.experimental.pallas import tpu as pltpu
from jax.experimental.pallas import tpu_sc as plsc

_NUM_CCSK = 8
_SEQ = 128
_F = _NUM_CCSK * _SEQ
_NC = 2
_NS = 16
_NW = _NC * _NS
_CH = 16


def _sc_body(x_hbm, out_hbm, in_v, out_v):
    wid = lax.axis_index("s") * _NC + lax.axis_index("c")
    rows_per_w = x_hbm.shape[0] // _F // _NW
    base = wid * rows_per_w

    def chunk_body(i, _):
        r0 = base + i * _CH
        pltpu.sync_copy(x_hbm.at[pl.ds(r0 * _F, _CH * _F)], in_v)

        def row_body(rr, _):
            off = rr * _F

            def col_body(c, _):
                lane = lax.iota(jnp.int32, 16)
                pat = ((lane & 7) << 7) | (lane >> 3)
                v = plsc.load_gather(in_v, [off + 2 * c + pat])
                out_v[pl.ds(off + c * 16, 16)] = v
                return 0

            return lax.fori_loop(0, _F // 16, col_body, 0)

        lax.fori_loop(0, _CH, row_body, 0)
        pltpu.sync_copy(out_v, out_hbm.at[pl.ds(r0 * _F, _CH * _F)])
        return 0

    lax.fori_loop(0, rows_per_w // _CH, chunk_body, 0)


def kernel(inputs):
    b, t, f = inputs.shape
    rows = b * t
    x = inputs.reshape(rows * f)
    mesh = plsc.VectorSubcoreMesh(core_axis_name="c", subcore_axis_name="s")
    k = functools.partial(
        pl.kernel,
        out_type=jax.ShapeDtypeStruct((rows * f,), jnp.float32),
        mesh=mesh,
        scratch_types=[
            pltpu.VMEM((_CH * f,), jnp.float32),
            pltpu.VMEM((_CH * f,), jnp.float32),
        ],
        compiler_params=pltpu.CompilerParams(needs_layout_passes=False),
    )(_sc_body)
    out = k(x)
    return out.reshape(b, t, f)


# speedup vs baseline: 1.6919x; 1.1192x over previous
import functools

import jax
import jax.numpy as jnp
from jax import lax
from jax.experimental import pallas as pl
from jax.experimental.pallas import tpu as pltpu
from jax.experimental.pallas import tpu_sc as plsc

_NUM_CCSK = 8
_SEQ = 128
_F = _NUM_CCSK * _SEQ
_NC = 2
_NS = 16
_NW = _NC * _NS
_CH = 16


def _permute_chunk(in_v, out_v):
    def row_body(rr, _):
        off = rr * _F
        lane = lax.iota(jnp.int32, 16)
        pat = ((lane & 7) << 7) | (lane >> 3)

        def col_body(c, _):
            v = plsc.load_gather(in_v, [off + 2 * c + pat])
            out_v[pl.ds(off + c * 16, 16)] = v
            return 0

        return lax.fori_loop(0, _F // 16, col_body, 0, unroll=8)

    lax.fori_loop(0, _CH, row_body, 0)


def _sc_body(x_hbm, out_hbm, in_a, in_b, out_a, out_b, si_a, si_b, so_a, so_b):
    wid = lax.axis_index("s") * _NC + lax.axis_index("c")
    rows_per_w = x_hbm.shape[0] // _F // _NW
    base = wid * rows_per_w
    n = rows_per_w // _CH

    ins = [in_a, in_b]
    outs = [out_a, out_b]
    sem_in = [si_a, si_b]
    sem_out = [so_a, so_b]

    def start_in(i):
        r0 = base + i * _CH
        return pltpu.async_copy(
            x_hbm.at[pl.ds(r0 * _F, _CH * _F)], ins[i % 2], sem_in[i % 2]
        )

    def start_out(i):
        r0 = base + i * _CH
        return pltpu.async_copy(
            outs[i % 2], out_hbm.at[pl.ds(r0 * _F, _CH * _F)], sem_out[i % 2]
        )

    in_descs = {0: start_in(0), 1: start_in(1)}
    out_descs = {}
    for i in range(n):
        in_descs.pop(i).wait()
        if i >= 2:
            out_descs.pop(i - 2).wait()
        _permute_chunk(ins[i % 2], outs[i % 2])
        out_descs[i] = start_out(i)
        if i + 2 < n:
            in_descs[i + 2] = start_in(i + 2)
    out_descs.pop(n - 2).wait()
    out_descs.pop(n - 1).wait()


def kernel(inputs):
    b, t, f = inputs.shape
    rows = b * t
    x = inputs.reshape(rows * f)
    mesh = plsc.VectorSubcoreMesh(core_axis_name="c", subcore_axis_name="s")
    k = functools.partial(
        pl.kernel,
        out_type=jax.ShapeDtypeStruct((rows * f,), jnp.float32),
        mesh=mesh,
        scratch_types=[
            pltpu.VMEM((_CH * f,), jnp.float32),
            pltpu.VMEM((_CH * f,), jnp.float32),
            pltpu.VMEM((_CH * f,), jnp.float32),
            pltpu.VMEM((_CH * f,), jnp.float32),
            pltpu.SemaphoreType.DMA,
            pltpu.SemaphoreType.DMA,
            pltpu.SemaphoreType.DMA,
            pltpu.SemaphoreType.DMA,
        ],
        compiler_params=pltpu.CompilerParams(needs_layout_passes=False),
    )(_sc_body)
    out = k(x)
    return out.reshape(b, t, f)
